# Initial kernel scaffold; baseline (speedup 1.0000x reference)
import jax
import jax.numpy as jnp
from jax.experimental import pallas as pl


def kernel(feature_v, edge_index, feature_e, trans_edge_index, Wv1, bv1, Wv2, bv2, Wv3, bv3, We1, be1, We2, be2, We3, be3):
    raise NotImplementedError("write your pallas kernel here")


# speedup vs baseline: 1.9555x; 1.9555x over previous
import jax
import jax.numpy as jnp
from jax.experimental import pallas as pl


def _matmul_kernel(x_ref, w_ref, o_ref):
    o_ref[...] = jnp.dot(x_ref[...], w_ref[...],
                         preferred_element_type=jnp.float32,
                         precision=jax.lax.Precision.HIGHEST)


def _pallas_matmul(x, w, block_rows=1024):
    n, k = x.shape
    m = w.shape[1]
    return pl.pallas_call(
        _matmul_kernel,
        grid=(pl.cdiv(n, block_rows),),
        in_specs=[
            pl.BlockSpec((block_rows, k), lambda i: (i, 0)),
            pl.BlockSpec((k, m), lambda i: (0, 0)),
        ],
        out_specs=pl.BlockSpec((block_rows, m), lambda i: (i, 0)),
        out_shape=jax.ShapeDtypeStruct((n, m), jnp.float32),
    )(x, w)


def _bias_relu_kernel(x_ref, b_ref, o_ref):
    o_ref[...] = jnp.maximum(x_ref[...] + b_ref[...], 0.0)


def _pallas_bias_relu(x, b, block_rows=2048):
    n, m = x.shape
    return pl.pallas_call(
        _bias_relu_kernel,
        grid=(pl.cdiv(n, block_rows),),
        in_specs=[
            pl.BlockSpec((block_rows, m), lambda i: (i, 0)),
            pl.BlockSpec((1, m), lambda i: (0, 0)),
        ],
        out_specs=pl.BlockSpec((block_rows, m), lambda i: (i, 0)),
        out_shape=jax.ShapeDtypeStruct((n, m), jnp.float32),
    )(x, b.reshape(1, m))


def _gcn_stack(x, edge_index, layers):
    n = x.shape[0]
    loop = jnp.arange(n, dtype=edge_index.dtype)
    src = jnp.concatenate([edge_index[0], loop])
    dst = jnp.concatenate([edge_index[1], loop])
    deg = jnp.zeros((n,), jnp.float32).at[dst].add(1.0)
    dinv = jnp.where(deg > 0, jax.lax.rsqrt(deg), 0.0)
    norm = dinv[src] * dinv[dst]
    for W, b in layers:
        xw = _pallas_matmul(x, W)
        msg = xw[src] * norm[:, None]
        agg = jnp.zeros((n, W.shape[1]), jnp.float32).at[dst].add(msg)
        x = _pallas_bias_relu(agg, b)
    return x


def kernel(feature_v, edge_index, feature_e, trans_edge_index,
           Wv1, bv1, Wv2, bv2, Wv3, bv3,
           We1, be1, We2, be2, We3, be3):
    fv = _gcn_stack(feature_v, edge_index,
                    [(Wv1, bv1), (Wv2, bv2), (Wv3, bv3)])
    fe = _gcn_stack(feature_e, trans_edge_index,
                    [(We1, be1), (We2, be2), (We3, be3)])
    return (fv, fe)


# speedup vs baseline: 5.8131x; 2.9727x over previous
import jax
import jax.numpy as jnp
from jax.experimental import pallas as pl


def _matmul_kernel(x_ref, w_ref, o_ref):
    o_ref[...] = jnp.dot(x_ref[...], w_ref[...],
                         preferred_element_type=jnp.float32,
                         precision=jax.lax.Precision.HIGHEST)


def _pallas_matmul(x, w, block_rows=1024):
    n, k = x.shape
    m = w.shape[1]
    return pl.pallas_call(
        _matmul_kernel,
        grid=(pl.cdiv(n, block_rows),),
        in_specs=[
            pl.BlockSpec((block_rows, k), lambda i: (i, 0)),
            pl.BlockSpec((k, m), lambda i: (0, 0)),
        ],
        out_specs=pl.BlockSpec((block_rows, m), lambda i: (i, 0)),
        out_shape=jax.ShapeDtypeStruct((n, m), jnp.float32),
    )(x, w)


def _scale_bias_relu_kernel(x_ref, d_ref, b_ref, o_ref):
    o_ref[...] = jnp.maximum(x_ref[...] * d_ref[...] + b_ref[...], 0.0)


def _pallas_scale_bias_relu(x, dcol, b, block_rows=2048):
    n, m = x.shape
    return pl.pallas_call(
        _scale_bias_relu_kernel,
        grid=(pl.cdiv(n, block_rows),),
        in_specs=[
            pl.BlockSpec((block_rows, m), lambda i: (i, 0)),
            pl.BlockSpec((block_rows, 1), lambda i: (i, 0)),
            pl.BlockSpec((1, m), lambda i: (0, 0)),
        ],
        out_specs=pl.BlockSpec((block_rows, m), lambda i: (i, 0)),
        out_shape=jax.ShapeDtypeStruct((n, m), jnp.float32),
    )(x, dcol, b.reshape(1, m))


def _gcn_stack(x, edge_index, layers):
    n = x.shape[0]
    src = edge_index[0]
    dst = edge_index[1]
    deg = jnp.ones((n,), jnp.float32).at[dst].add(1.0)
    dinv = jax.lax.rsqrt(deg)
    dcol = dinv[:, None]
    for W, b in layers:
        xw = _pallas_matmul(x, W)
        y = xw * dcol
        agg = y.at[dst].add(y[src])
        x = _pallas_scale_bias_relu(agg, dcol, b)
    return x


def kernel(feature_v, edge_index, feature_e, trans_edge_index,
           Wv1, bv1, Wv2, bv2, Wv3, bv3,
           We1, be1, We2, be2, We3, be3):
    fv = _gcn_stack(feature_v, edge_index,
                    [(Wv1, bv1), (Wv2, bv2), (Wv3, bv3)])
    fe = _gcn_stack(feature_e, trans_edge_index,
                    [(We1, be1), (We2, be2), (We3, be3)])
    return (fv, fe)
